# baseline (speedup 1.0000x reference)
import functools

import jax
import jax.numpy as jnp
import numpy as np
from jax.experimental import pallas as pl
from jax.experimental.pallas import tpu as pltpu

_BN_EPS = 1e-4


def _bn(x, p):
    return (x - p["rm"]) / jnp.sqrt(p["rv"] + _BN_EPS) * p["gamma"] + p["beta"]


def _conv_bn_relu(x, p):
    return jax.nn.relu(_bn(x @ p["W"] + p["b"], p))


def _square_distance(src, dst):
    return (jnp.sum(src ** 2, -1)[:, :, None] + jnp.sum(dst ** 2, -1)[:, None, :]
            - 2.0 * jnp.einsum("bnc,bmc->bnm", src, dst))


def _index_points(points, idx):
    return jax.vmap(lambda p, i: p[i])(points, idx)


def _fps(xyz, npoint):
    B, N, _ = xyz.shape

    def body(i, state):
        centroids, distance, farthest = state
        centroids = centroids.at[:, i].set(farthest)
        centroid = _index_points(xyz, farthest[:, None])
        dist = jnp.sum((xyz - centroid) ** 2, -1)
        distance = jnp.minimum(distance, dist)
        farthest = jnp.argmax(distance, axis=-1).astype(jnp.int32)
        return centroids, distance, farthest

    centroids = jnp.zeros((B, npoint), dtype=jnp.int32)
    distance = jnp.full((B, N), 1e10, dtype=jnp.float32)
    farthest = jnp.zeros((B,), dtype=jnp.int32)
    centroids, _, _ = jax.lax.fori_loop(0, npoint, body, (centroids, distance, farthest))
    return centroids


def _ball_query(radius, nsample, xyz, new_xyz):
    B, N, _ = xyz.shape
    S = new_xyz.shape[1]
    sqrdists = _square_distance(new_xyz, xyz)
    idx = jnp.broadcast_to(jnp.arange(N, dtype=jnp.int32), (B, S, N))
    idx = jnp.where(sqrdists > radius ** 2, N, idx)
    neg_vals, _ = jax.lax.top_k(-idx, nsample)
    group_idx = -neg_vals
    first = group_idx[:, :, :1]
    group_idx = jnp.where(group_idx == N, jnp.broadcast_to(first, group_idx.shape), group_idx)
    return jnp.minimum(group_idx, N - 1)


def _sa(xyz, points, npoint, radius, nsample, mlp):
    xyz_sg = jax.lax.stop_gradient(xyz)
    fps_idx = _fps(xyz_sg, npoint)
    new_xyz = _index_points(xyz, fps_idx)
    idx = _ball_query(radius, nsample, xyz_sg, jax.lax.stop_gradient(new_xyz))
    grouped_xyz = _index_points(xyz, idx) - new_xyz[:, :, None, :]
    if points is not None:
        new_points = jnp.concatenate([grouped_xyz, _index_points(points, idx)], axis=-1)
    else:
        new_points = grouped_xyz
    x = new_points
    for p in mlp:
        x = _conv_bn_relu(x, p)
    return new_xyz, jnp.max(x, axis=2)


def _fp(xyz1, xyz2, points1, points2, mlp):
    dists = _square_distance(xyz1, xyz2)
    neg, idx = jax.lax.top_k(-dists, 3)
    d = -neg
    dist_recip = 1.0 / (d + 1e-8)
    weight = dist_recip / jnp.sum(dist_recip, axis=2, keepdims=True)
    interpolated = jnp.sum(_index_points(points2, idx) * weight[..., None], axis=2)
    new_points = interpolated if points1 is None else jnp.concatenate([points1, interpolated], axis=-1)
    x = new_points
    for p in mlp:
        x = _conv_bn_relu(x, p)
    return x


def _fold_bn(p):
    inv = 1.0 / jnp.sqrt(p["rv"] + _BN_EPS)
    scale = p["gamma"] * inv
    shift = p["beta"] - p["rm"] * inv * p["gamma"]
    return p["W"], p["b"], scale, shift


def _heads_kernel(bb_ref, w_sh_ref, b_sh_ref, w_so_ref, b_so_ref,
                  w_oh_ref, b_oh_ref, w_oo_ref, b_oo_ref,
                  sem_ref, off_ref):
    x = bb_ref[0]
    h = jnp.maximum(x @ w_sh_ref[...] + b_sh_ref[...], 0.0)
    sem_ref[0] = h @ w_so_ref[...] + b_so_ref[...]
    h2 = jnp.maximum(x @ w_oh_ref[...] + b_oh_ref[...], 0.0)
    off_ref[0] = h2 @ w_oo_ref[...] + b_oo_ref[...]


def _heads(bb, params):
    B, N, C = bb.shape
    W_sh, b_sh, sc_sh, sh_sh = _fold_bn(params["sem_hidden"])
    W_sh2 = W_sh * sc_sh[None, :]
    b_sh2 = b_sh * sc_sh + sh_sh
    W_oh, b_oh, sc_oh, sh_oh = _fold_bn(params["off_hidden"])
    W_oh2 = W_oh * sc_oh[None, :]
    b_oh2 = b_oh * sc_oh + sh_oh
    W_so = params["sem_out"]["W"]
    b_so = params["sem_out"]["b"]
    W_oo = params["off_out"]["W"]
    b_oo = params["off_out"]["b"]

    BLK = 2048
    grid = (B, N // BLK)
    sem, off = pl.pallas_call(
        _heads_kernel,
        grid=grid,
        in_specs=[
            pl.BlockSpec((1, BLK, C), lambda b, n: (b, n, 0)),
            pl.BlockSpec((C, C), lambda b, n: (0, 0)),
            pl.BlockSpec((C,), lambda b, n: (0,)),
            pl.BlockSpec((C, 2), lambda b, n: (0, 0)),
            pl.BlockSpec((2,), lambda b, n: (0,)),
            pl.BlockSpec((C, C), lambda b, n: (0, 0)),
            pl.BlockSpec((C,), lambda b, n: (0,)),
            pl.BlockSpec((C, 3), lambda b, n: (0, 0)),
            pl.BlockSpec((3,), lambda b, n: (0,)),
        ],
        out_specs=[
            pl.BlockSpec((1, BLK, 2), lambda b, n: (b, n, 0)),
            pl.BlockSpec((1, BLK, 3), lambda b, n: (b, n, 0)),
        ],
        out_shape=[
            jax.ShapeDtypeStruct((B, N, 2), jnp.float32),
            jax.ShapeDtypeStruct((B, N, 3), jnp.float32),
        ],
    )(bb, W_sh2, b_sh2, W_so, b_so, W_oh2, b_oh2, W_oo, b_oo)
    return sem, off


def kernel(coords, feats, params):
    l0_xyz = jnp.transpose(coords, (0, 2, 1))
    l0_points = jnp.transpose(feats, (0, 2, 1))
    l1_xyz, l1_points = _sa(l0_xyz, l0_points, 1024, 0.1, 32, params["sa1"])
    l2_xyz, l2_points = _sa(l1_xyz, l1_points, 256, 0.2, 32, params["sa2"])
    l3_xyz, l3_points = _sa(l2_xyz, l2_points, 64, 0.4, 32, params["sa3"])
    l4_xyz, l4_points = _sa(l3_xyz, l3_points, 16, 0.8, 32, params["sa4"])
    l3_points = _fp(l3_xyz, l4_xyz, l3_points, l4_points, params["fp4"])
    l2_points = _fp(l2_xyz, l3_xyz, l2_points, l3_points, params["fp3"])
    l1_points = _fp(l1_xyz, l2_xyz, l1_points, l2_points, params["fp2"])
    l0_points = _fp(l0_xyz, l1_xyz, None, l1_points, params["fp1"])
    sem, off = _heads(l0_points, params)
    backbone_feats = jnp.transpose(l0_points, (0, 2, 1))
    semantic_prediction_logits = jnp.transpose(sem, (0, 2, 1))
    offset_predictions = jnp.transpose(off, (0, 2, 1))
    return backbone_feats, semantic_prediction_logits, offset_predictions


# speedup vs baseline: 1.5593x; 1.5593x over previous
import functools

import jax
import jax.numpy as jnp
import numpy as np
from jax.experimental import pallas as pl
from jax.experimental.pallas import tpu as pltpu

_BN_EPS = 1e-4


def _bn(x, p):
    return (x - p["rm"]) / jnp.sqrt(p["rv"] + _BN_EPS) * p["gamma"] + p["beta"]


def _conv_bn_relu(x, p):
    return jax.nn.relu(_bn(x @ p["W"] + p["b"], p))


def _square_distance(src, dst):
    return (jnp.sum(src ** 2, -1)[:, :, None] + jnp.sum(dst ** 2, -1)[:, None, :]
            - 2.0 * jnp.einsum("bnc,bmc->bnm", src, dst))


def _index_points(points, idx):
    return jax.vmap(lambda p, i: p[i])(points, idx)


_FPS_SIZES = (1024, 256, 64, 16)


def _fps_kernel(xyz_ref, o1_ref, o2_ref, o3_ref, o4_ref):
    x = xyz_ref[0, 0]
    y = xyz_ref[0, 1]
    z = xyz_ref[0, 2]
    out_refs = (o1_ref, o2_ref, o3_ref, o4_ref)
    for p, out_ref in zip(_FPS_SIZES, out_refs):
        m = x.shape[1]
        s8 = p // 8
        idx2d = (jax.lax.broadcasted_iota(jnp.int32, (8, m), 0) * m
                 + jax.lax.broadcasted_iota(jnp.int32, (8, m), 1))
        oidx2d = (jax.lax.broadcasted_iota(jnp.int32, (8, s8), 0) * s8
                  + jax.lax.broadcasted_iota(jnp.int32, (8, s8), 1))

        def body(i, st, x=x, y=y, z=z, idx2d=idx2d, oidx2d=oidx2d):
            dist, far, nx, ny, nz = st
            sel = idx2d == far
            cx = jnp.sum(jnp.where(sel, x, 0.0))
            cy = jnp.sum(jnp.where(sel, y, 0.0))
            cz = jnp.sum(jnp.where(sel, z, 0.0))
            oh = oidx2d == i
            nx = jnp.where(oh, cx, nx)
            ny = jnp.where(oh, cy, ny)
            nz = jnp.where(oh, cz, nz)
            d = (x - cx) ** 2 + (y - cy) ** 2
            d = d + (z - cz) ** 2
            dist = jnp.minimum(dist, d)
            mx = jnp.max(dist)
            cand = jnp.where(dist == mx, idx2d, jnp.int32(2 ** 30))
            far = jnp.min(cand)
            return dist, far, nx, ny, nz

        init = (jnp.full((8, m), 1e10, jnp.float32), jnp.int32(0),
                jnp.zeros((8, s8), jnp.float32), jnp.zeros((8, s8), jnp.float32),
                jnp.zeros((8, s8), jnp.float32))
        _, _, nx, ny, nz = jax.lax.fori_loop(0, p, body, init)
        out_ref[0, 0] = nx
        out_ref[0, 1] = ny
        out_ref[0, 2] = nz
        x, y, z = nx, ny, nz


def _fps_all(xyz_t):
    B, _, N = xyz_t.shape
    xyz4 = xyz_t.reshape(B, 3, 8, N // 8)
    outs = pl.pallas_call(
        _fps_kernel,
        grid=(B,),
        in_specs=[pl.BlockSpec((1, 3, 8, N // 8), lambda b: (b, 0, 0, 0))],
        out_specs=[pl.BlockSpec((1, 3, 8, s // 8), lambda b: (b, 0, 0, 0))
                   for s in _FPS_SIZES],
        out_shape=[jax.ShapeDtypeStruct((B, 3, 8, s // 8), jnp.float32)
                   for s in _FPS_SIZES],
    )(xyz4)
    return [o.reshape(B, 3, -1).transpose(0, 2, 1) for o in outs]


def _ball_query(radius, nsample, xyz, new_xyz):
    B, N, _ = xyz.shape
    S = new_xyz.shape[1]
    sqrdists = _square_distance(new_xyz, xyz)
    idx = jnp.broadcast_to(jnp.arange(N, dtype=jnp.int32), (B, S, N))
    idx = jnp.where(sqrdists > radius ** 2, N, idx)
    neg_vals, _ = jax.lax.top_k(-idx, nsample)
    group_idx = -neg_vals
    first = group_idx[:, :, :1]
    group_idx = jnp.where(group_idx == N, jnp.broadcast_to(first, group_idx.shape), group_idx)
    return jnp.minimum(group_idx, N - 1)


def _sa(xyz, points, new_xyz, radius, nsample, mlp):
    idx = _ball_query(radius, nsample, xyz, new_xyz)
    grouped_xyz = _index_points(xyz, idx) - new_xyz[:, :, None, :]
    if points is not None:
        new_points = jnp.concatenate([grouped_xyz, _index_points(points, idx)], axis=-1)
    else:
        new_points = grouped_xyz
    x = new_points
    for p in mlp:
        x = _conv_bn_relu(x, p)
    return new_xyz, jnp.max(x, axis=2)


def _fp(xyz1, xyz2, points1, points2, mlp):
    dists = _square_distance(xyz1, xyz2)
    neg, idx = jax.lax.top_k(-dists, 3)
    d = -neg
    dist_recip = 1.0 / (d + 1e-8)
    weight = dist_recip / jnp.sum(dist_recip, axis=2, keepdims=True)
    interpolated = jnp.sum(_index_points(points2, idx) * weight[..., None], axis=2)
    new_points = interpolated if points1 is None else jnp.concatenate([points1, interpolated], axis=-1)
    x = new_points
    for p in mlp:
        x = _conv_bn_relu(x, p)
    return x


def _fold_bn(p):
    inv = 1.0 / jnp.sqrt(p["rv"] + _BN_EPS)
    scale = p["gamma"] * inv
    shift = p["beta"] - p["rm"] * inv * p["gamma"]
    return p["W"], p["b"], scale, shift


def _heads_kernel(bb_ref, w_sh_ref, b_sh_ref, w_so_ref, b_so_ref,
                  w_oh_ref, b_oh_ref, w_oo_ref, b_oo_ref,
                  sem_ref, off_ref):
    x = bb_ref[0]
    h = jnp.maximum(x @ w_sh_ref[...] + b_sh_ref[...], 0.0)
    sem_ref[0] = h @ w_so_ref[...] + b_so_ref[...]
    h2 = jnp.maximum(x @ w_oh_ref[...] + b_oh_ref[...], 0.0)
    off_ref[0] = h2 @ w_oo_ref[...] + b_oo_ref[...]


def _heads(bb, params):
    B, N, C = bb.shape
    W_sh, b_sh, sc_sh, sh_sh = _fold_bn(params["sem_hidden"])
    W_sh2 = W_sh * sc_sh[None, :]
    b_sh2 = b_sh * sc_sh + sh_sh
    W_oh, b_oh, sc_oh, sh_oh = _fold_bn(params["off_hidden"])
    W_oh2 = W_oh * sc_oh[None, :]
    b_oh2 = b_oh * sc_oh + sh_oh
    W_so = params["sem_out"]["W"]
    b_so = params["sem_out"]["b"]
    W_oo = params["off_out"]["W"]
    b_oo = params["off_out"]["b"]

    BLK = 2048
    grid = (B, N // BLK)
    sem, off = pl.pallas_call(
        _heads_kernel,
        grid=grid,
        in_specs=[
            pl.BlockSpec((1, BLK, C), lambda b, n: (b, n, 0)),
            pl.BlockSpec((C, C), lambda b, n: (0, 0)),
            pl.BlockSpec((C,), lambda b, n: (0,)),
            pl.BlockSpec((C, 2), lambda b, n: (0, 0)),
            pl.BlockSpec((2,), lambda b, n: (0,)),
            pl.BlockSpec((C, C), lambda b, n: (0, 0)),
            pl.BlockSpec((C,), lambda b, n: (0,)),
            pl.BlockSpec((C, 3), lambda b, n: (0, 0)),
            pl.BlockSpec((3,), lambda b, n: (0,)),
        ],
        out_specs=[
            pl.BlockSpec((1, BLK, 2), lambda b, n: (b, n, 0)),
            pl.BlockSpec((1, BLK, 3), lambda b, n: (b, n, 0)),
        ],
        out_shape=[
            jax.ShapeDtypeStruct((B, N, 2), jnp.float32),
            jax.ShapeDtypeStruct((B, N, 3), jnp.float32),
        ],
    )(bb, W_sh2, b_sh2, W_so, b_so, W_oh2, b_oh2, W_oo, b_oo)
    return sem, off


def kernel(coords, feats, params):
    l0_xyz = jnp.transpose(coords, (0, 2, 1))
    l0_points = jnp.transpose(feats, (0, 2, 1))
    nx1, nx2, nx3, nx4 = _fps_all(coords)
    l1_xyz, l1_points = _sa(l0_xyz, l0_points, nx1, 0.1, 32, params["sa1"])
    l2_xyz, l2_points = _sa(l1_xyz, l1_points, nx2, 0.2, 32, params["sa2"])
    l3_xyz, l3_points = _sa(l2_xyz, l2_points, nx3, 0.4, 32, params["sa3"])
    l4_xyz, l4_points = _sa(l3_xyz, l3_points, nx4, 0.8, 32, params["sa4"])
    l3_points = _fp(l3_xyz, l4_xyz, l3_points, l4_points, params["fp4"])
    l2_points = _fp(l2_xyz, l3_xyz, l2_points, l3_points, params["fp3"])
    l1_points = _fp(l1_xyz, l2_xyz, l1_points, l2_points, params["fp2"])
    l0_points = _fp(l0_xyz, l1_xyz, None, l1_points, params["fp1"])
    sem, off = _heads(l0_points, params)
    backbone_feats = jnp.transpose(l0_points, (0, 2, 1))
    semantic_prediction_logits = jnp.transpose(sem, (0, 2, 1))
    offset_predictions = jnp.transpose(off, (0, 2, 1))
    return backbone_feats, semantic_prediction_logits, offset_predictions


# speedup vs baseline: 5.0772x; 3.2560x over previous
import functools

import jax
import jax.numpy as jnp
import numpy as np
from jax.experimental import pallas as pl
from jax.experimental.pallas import tpu as pltpu

_BN_EPS = 1e-4


def _bn(x, p):
    return (x - p["rm"]) / jnp.sqrt(p["rv"] + _BN_EPS) * p["gamma"] + p["beta"]


def _conv_bn_relu(x, p):
    return jax.nn.relu(_bn(x @ p["W"] + p["b"], p))


def _square_distance(src, dst):
    return (jnp.sum(src ** 2, -1)[:, :, None] + jnp.sum(dst ** 2, -1)[:, None, :]
            - 2.0 * jnp.einsum("bnc,bmc->bnm", src, dst))


def _index_points(points, idx):
    return jax.vmap(lambda p, i: p[i])(points, idx)


_FPS_SIZES = (1024, 256, 64, 16)


def _fps_kernel(xyz_ref, o1_ref, o2_ref, o3_ref, o4_ref):
    x = xyz_ref[0, 0]
    y = xyz_ref[0, 1]
    z = xyz_ref[0, 2]
    out_refs = (o1_ref, o2_ref, o3_ref, o4_ref)
    for p, out_ref in zip(_FPS_SIZES, out_refs):
        m = x.shape[1]
        s8 = p // 8
        idx2d = (jax.lax.broadcasted_iota(jnp.int32, (8, m), 0) * m
                 + jax.lax.broadcasted_iota(jnp.int32, (8, m), 1))
        oidx2d = (jax.lax.broadcasted_iota(jnp.int32, (8, s8), 0) * s8
                  + jax.lax.broadcasted_iota(jnp.int32, (8, s8), 1))

        def body(i, st, x=x, y=y, z=z, idx2d=idx2d, oidx2d=oidx2d):
            dist, far, nx, ny, nz = st
            sel = idx2d == far
            cx = jnp.sum(jnp.where(sel, x, 0.0))
            cy = jnp.sum(jnp.where(sel, y, 0.0))
            cz = jnp.sum(jnp.where(sel, z, 0.0))
            oh = oidx2d == i
            nx = jnp.where(oh, cx, nx)
            ny = jnp.where(oh, cy, ny)
            nz = jnp.where(oh, cz, nz)
            d = (x - cx) ** 2 + (y - cy) ** 2
            d = d + (z - cz) ** 2
            dist = jnp.minimum(dist, d)
            mx = jnp.max(dist)
            cand = jnp.where(dist == mx, idx2d, jnp.int32(2 ** 30))
            far = jnp.min(cand)
            return dist, far, nx, ny, nz

        init = (jnp.full((8, m), 1e10, jnp.float32), jnp.int32(0),
                jnp.zeros((8, s8), jnp.float32), jnp.zeros((8, s8), jnp.float32),
                jnp.zeros((8, s8), jnp.float32))
        _, _, nx, ny, nz = jax.lax.fori_loop(0, p, body, init)
        out_ref[0, 0] = nx
        out_ref[0, 1] = ny
        out_ref[0, 2] = nz
        x, y, z = nx, ny, nz


def _fps_all(xyz_t):
    B, _, N = xyz_t.shape
    xyz4 = xyz_t.reshape(B, 3, 8, N // 8)
    outs = pl.pallas_call(
        _fps_kernel,
        grid=(B,),
        in_specs=[pl.BlockSpec((1, 3, 8, N // 8), lambda b: (b, 0, 0, 0))],
        out_specs=[pl.BlockSpec((1, 3, 8, s // 8), lambda b: (b, 0, 0, 0))
                   for s in _FPS_SIZES],
        out_shape=[jax.ShapeDtypeStruct((B, 3, 8, s // 8), jnp.float32)
                   for s in _FPS_SIZES],
    )(xyz4)
    return [o.reshape(B, 3, -1).transpose(0, 2, 1) for o in outs]


def _ball_query(radius, nsample, xyz, new_xyz):
    B, N, _ = xyz.shape
    S = new_xyz.shape[1]
    sqrdists = _square_distance(new_xyz, xyz)
    mask = sqrdists <= radius ** 2
    CH = min(128, N)
    mf = mask.astype(jnp.float32).reshape(B, S, N // CH, CH)
    tri = (jnp.arange(CH)[:, None] <= jnp.arange(CH)[None, :]).astype(jnp.float32)
    inner = jnp.einsum("bsck,kl->bscl", mf, tri)
    chunk = jnp.sum(mf, -1)
    carry = jnp.cumsum(chunk, -1) - chunk
    cc = (inner + carry[..., None]).reshape(B, S, N)
    q = jnp.arange(1, nsample + 1, dtype=jnp.float32)
    gi = jax.vmap(lambda row: jnp.searchsorted(row, q, side="left"))(
        cc.reshape(B * S, N)).reshape(B, S, nsample).astype(jnp.int32)
    first = gi[:, :, :1]
    gi = jnp.where(gi == N, jnp.broadcast_to(first, gi.shape), gi)
    return jnp.minimum(gi, N - 1)


def _sa(xyz, points, new_xyz, radius, nsample, mlp):
    idx = _ball_query(radius, nsample, xyz, new_xyz)
    grouped_xyz = _index_points(xyz, idx) - new_xyz[:, :, None, :]
    if points is not None:
        new_points = jnp.concatenate([grouped_xyz, _index_points(points, idx)], axis=-1)
    else:
        new_points = grouped_xyz
    x = new_points
    for p in mlp:
        x = _conv_bn_relu(x, p)
    return new_xyz, jnp.max(x, axis=2)


def _fold_mlp(mlp):
    out = []
    for p in mlp:
        inv = 1.0 / jnp.sqrt(p["rv"] + _BN_EPS)
        scale = p["gamma"] * inv
        shift = p["beta"] - p["rm"] * inv * p["gamma"]
        out.append((p["W"] * scale[None, :], p["b"] * scale + shift))
    return out


def _fp(xyz1, xyz2, points1, points2, mlp, heads=None, rows=None):
    B, n, _ = xyz1.shape
    m = xyz2.shape[1]
    c2 = points2.shape[2]
    c1 = 0 if points1 is None else points1.shape[2]
    ws = _fold_mlp(mlp)
    rows = rows or n
    xyz2t = jnp.transpose(xyz2, (0, 2, 1))

    n_w = len(ws)

    def kern(*refs):
        x1_ref, x2_ref, p2_ref = refs[0], refs[1], refs[2]
        i = 3
        p1_ref = None
        if c1:
            p1_ref = refs[i]
            i += 1
        w_refs = refs[i:i + 2 * n_w]
        i += 2 * n_w
        if heads is not None:
            h_refs = refs[i:i + 8]
            i += 8
        out_refs = refs[i:]

        x1 = x1_ref[0]
        x2t = x2_ref[0]
        s2 = jnp.sum(x1 * x1, axis=1, keepdims=True)
        d2 = jnp.sum(x2t * x2t, axis=0, keepdims=True)
        dots = jnp.dot(x1, x2t, preferred_element_type=jnp.float32)
        dist = s2 + d2 - 2.0 * dots
        iota = jax.lax.broadcasted_iota(jnp.int32, dist.shape, 1)
        d0 = dist
        vs, js = [], []
        for _ in range(3):
            v = jnp.min(d0, axis=1, keepdims=True)
            j = jnp.min(jnp.where(d0 == v, iota, jnp.int32(m)), axis=1,
                        keepdims=True)
            vs.append(v)
            js.append(j)
            d0 = jnp.where(iota == j, jnp.float32(jnp.inf), d0)
        r0, r1, r2 = (1.0 / (v + 1e-8) for v in vs)
        norm = (r0 + r1) + r2
        wm = jnp.zeros(dist.shape, jnp.float32)
        for r, j in zip((r0, r1, r2), js):
            wm = wm + jnp.where(iota == j, r / norm, 0.0)
        interp = jnp.dot(wm, p2_ref[0], preferred_element_type=jnp.float32)

        w0, b0 = w_refs[0][...], w_refs[1][...]
        acc = jnp.dot(interp, w0[c1:], preferred_element_type=jnp.float32)
        if c1:
            acc = jnp.dot(p1_ref[0], w0[:c1],
                          preferred_element_type=jnp.float32) + acc
        x = jnp.maximum(acc + b0, 0.0)
        for li in range(1, n_w):
            w, b = w_refs[2 * li][...], w_refs[2 * li + 1][...]
            x = jnp.maximum(jnp.dot(x, w, preferred_element_type=jnp.float32)
                            + b, 0.0)
        out_refs[0][0] = x
        if heads is not None:
            wsh, bsh, wso, bso, woh, boh, woo, boo = (r[...] for r in h_refs)
            hs = jnp.maximum(jnp.dot(x, wsh, preferred_element_type=jnp.float32) + bsh, 0.0)
            out_refs[1][0] = jnp.dot(hs, wso, preferred_element_type=jnp.float32) + bso
            ho = jnp.maximum(jnp.dot(x, woh, preferred_element_type=jnp.float32) + boh, 0.0)
            out_refs[2][0] = jnp.dot(ho, woo, preferred_element_type=jnp.float32) + boo

    grid = (B, n // rows)
    in_specs = [
        pl.BlockSpec((1, rows, 3), lambda b, i: (b, i, 0)),
        pl.BlockSpec((1, 3, m), lambda b, i: (b, 0, 0)),
        pl.BlockSpec((1, m, c2), lambda b, i: (b, 0, 0)),
    ]
    args = [xyz1, xyz2t, points2]
    if c1:
        in_specs.append(pl.BlockSpec((1, rows, c1), lambda b, i: (b, i, 0)))
        args.append(points1)
    for w, b in ws:
        in_specs.append(pl.BlockSpec(w.shape, lambda b, i: (0, 0)))
        in_specs.append(pl.BlockSpec(b.shape, lambda b, i: (0,)))
        args.extend((w, b))
    cout = ws[-1][0].shape[1]
    out_specs = [pl.BlockSpec((1, rows, cout), lambda b, i: (b, i, 0))]
    out_shape = [jax.ShapeDtypeStruct((B, n, cout), jnp.float32)]
    if heads is not None:
        for h in heads:
            in_specs.append(pl.BlockSpec(h.shape,
                                         (lambda b, i: (0, 0)) if h.ndim == 2
                                         else (lambda b, i: (0,))))
            args.append(h)
        out_specs += [pl.BlockSpec((1, rows, 2), lambda b, i: (b, i, 0)),
                      pl.BlockSpec((1, rows, 3), lambda b, i: (b, i, 0))]
        out_shape += [jax.ShapeDtypeStruct((B, n, 2), jnp.float32),
                      jax.ShapeDtypeStruct((B, n, 3), jnp.float32)]
    res = pl.pallas_call(kern, grid=grid, in_specs=in_specs,
                         out_specs=out_specs, out_shape=out_shape)(*args)
    return res if heads is not None else res[0]


def kernel(coords, feats, params):
    l0_xyz = jnp.transpose(coords, (0, 2, 1))
    l0_points = jnp.transpose(feats, (0, 2, 1))
    nx1, nx2, nx3, nx4 = _fps_all(coords)
    l1_xyz, l1_points = _sa(l0_xyz, l0_points, nx1, 0.1, 32, params["sa1"])
    l2_xyz, l2_points = _sa(l1_xyz, l1_points, nx2, 0.2, 32, params["sa2"])
    l3_xyz, l3_points = _sa(l2_xyz, l2_points, nx3, 0.4, 32, params["sa3"])
    l4_xyz, l4_points = _sa(l3_xyz, l3_points, nx4, 0.8, 32, params["sa4"])
    l3_points = _fp(l3_xyz, l4_xyz, l3_points, l4_points, params["fp4"])
    l2_points = _fp(l2_xyz, l3_xyz, l2_points, l3_points, params["fp3"])
    l1_points = _fp(l1_xyz, l2_xyz, l1_points, l2_points, params["fp2"], rows=512)
    hsem = _fold_mlp([params["sem_hidden"]])[0]
    hoff = _fold_mlp([params["off_hidden"]])[0]
    heads = (hsem[0], hsem[1], params["sem_out"]["W"], params["sem_out"]["b"],
             hoff[0], hoff[1], params["off_out"]["W"], params["off_out"]["b"])
    l0_points, sem, off = _fp(l0_xyz, l1_xyz, None, l1_points, params["fp1"],
                              heads=heads, rows=512)
    backbone_feats = jnp.transpose(l0_points, (0, 2, 1))
    semantic_prediction_logits = jnp.transpose(sem, (0, 2, 1))
    offset_predictions = jnp.transpose(off, (0, 2, 1))
    return backbone_feats, semantic_prediction_logits, offset_predictions
